# Initial kernel scaffold; baseline (speedup 1.0000x reference)
import jax
import jax.numpy as jnp
from jax.experimental import pallas as pl


def kernel(freq, flit, op_type, W_freq, b_freq, W_flit, b_flit, W_op, b_op, W_fh, b_fh, W_fn, b_fn, Wr1, br1, Wp1, bp1, Wr2, br2, Wp2, bp2, Wh1, bh1, Wh2, bh2, Wh3, bh3, pass_src, pass_dst, transfer_src, transfer_dst, connect_src, connect_dst):
    raise NotImplementedError("write your pallas kernel here")


# speedup vs baseline: 1.7791x; 1.7791x over previous
import functools

import jax
import jax.numpy as jnp
from jax import lax
from jax.experimental import pallas as pl
from jax.experimental.pallas import tpu as pltpu
from jax.experimental.pallas import tpu_sc as plsc

H = 128
NC = 2
NS = 16
LN = 16
EB = 96

FP = jnp.float32


def _fill(ref, nrows, ncols, value):
    vals = jnp.full((LN,), value, FP)

    def body(i, c):
        for j in range(ncols // LN):
            ref[i, pl.ds(j * LN, LN)] = vals
        return c

    lax.fori_loop(0, nrows, body, 0)


def _round_robin(total, bs, nworkers, wid, fn):
    nf = total // bs
    rem = total - nf * bs
    kmax = (nf + nworkers - 1) // nworkers
    if kmax > 0:
        def body(k, c):
            b = wid + k * nworkers

            @pl.when(b < nf)
            def _():
                fn(b * bs, bs)

            return c

        lax.fori_loop(0, kmax, body, 0)
    if rem:
        @pl.when(wid == 0)
        def _():
            fn(nf * bs, rem)


def _make_seg_partials(n_dst, n_edges_padded, width):
    n_acc = n_dst + 8

    @functools.partial(
        pl.kernel,
        out_type=jax.ShapeDtypeStruct((NC, n_dst, width), FP),
        mesh=plsc.VectorSubcoreMesh(core_axis_name="c", subcore_axis_name="s",
                                    num_cores=NC, num_subcores=NS),
        scratch_types=[
            pltpu.VMEM((EB,), jnp.int32),
            pltpu.VMEM((EB,), jnp.int32),
            pltpu.VMEM((EB, width), FP),
            pltpu.VMEM_SHARED((n_acc, width), FP),
            pltpu.SemaphoreType.DMA,
        ],
        name=f"sc_seg_partials_{n_dst}_{n_edges_padded}_{width}",
    )
    def kern(table, src, dst, out, idx_s, idx_d, rows, acc, sem):
        cid = lax.axis_index("c")
        sid = lax.axis_index("s")
        wid = cid * NS + sid

        _fill(rows, EB, width, 0.0)

        def zero_blk(off, sz):
            pltpu.sync_copy(rows.at[pl.ds(0, sz)], acc.at[pl.ds(off, sz)])

        _round_robin(n_acc, EB, NS, sid, zero_blk)
        plsc.subcore_barrier()

        def edge_blk(off, sz):
            pltpu.sync_copy(src.at[pl.ds(off, EB)], idx_s)
            pltpu.sync_copy(dst.at[pl.ds(off, EB)], idx_d)
            pltpu.async_copy(table.at[idx_s], rows, sem).wait()
            pltpu.sync_copy(rows, acc.at[idx_d], add=True)

        _round_robin(n_edges_padded, EB, NC * NS, wid, edge_blk)
        plsc.subcore_barrier()

        def flush_blk(off, sz):
            pltpu.sync_copy(acc.at[pl.ds(off, sz)], out.at[cid, pl.ds(off, sz)])

        _round_robin(n_dst, EB, NS, sid, flush_blk)

    return kern


ND8 = 6256


def _make_cnt_grouped(n_edges_padded):
    n_acc = ND8 + 8

    @functools.partial(
        pl.kernel,
        out_type=jax.ShapeDtypeStruct((NC, ND8, H), FP),
        mesh=plsc.VectorSubcoreMesh(core_axis_name="c", subcore_axis_name="s",
                                    num_cores=NC, num_subcores=NS),
        scratch_types=[
            pltpu.VMEM((EB,), jnp.int32),
            pltpu.VMEM((EB,), jnp.int32),
            pltpu.VMEM((EB,), jnp.int32),
            pltpu.VMEM((EB, H), FP),
            pltpu.VMEM_SHARED((n_acc, H), FP),
            pltpu.SemaphoreType.DMA,
        ],
        name=f"sc_cnt_grouped_{n_edges_padded}",
    )
    def kern(onehot, dst, out, idx_d, gsrc, gdst, rows, acc, sem):
        cid = lax.axis_index("c")
        sid = lax.axis_index("s")
        wid = cid * NS + sid

        _fill(rows, EB, H, 0.0)

        def zero_blk(off, sz):
            pltpu.sync_copy(rows.at[pl.ds(0, sz)], acc.at[pl.ds(off, sz)])

        _round_robin(n_acc, EB, NS, sid, zero_blk)
        plsc.subcore_barrier()

        def edge_blk(off, sz):
            pltpu.sync_copy(dst.at[pl.ds(off, EB)], idx_d)
            for j in range(EB // LN):
                d = idx_d[pl.ds(j * LN, LN)]
                gsrc[pl.ds(j * LN, LN)] = d & 7
                gdst[pl.ds(j * LN, LN)] = jnp.minimum(d >> 3, ND8)
            pltpu.async_copy(onehot.at[gsrc], rows, sem).wait()
            pltpu.sync_copy(rows, acc.at[gdst], add=True)

        _round_robin(n_edges_padded, EB, NC * NS, wid, edge_blk)
        plsc.subcore_barrier()

        def flush_blk(off, sz):
            pltpu.sync_copy(acc.at[pl.ds(off, sz)], out.at[cid, pl.ds(off, sz)])

        _round_robin(ND8, EB, NS, sid, flush_blk)

    return kern


CH = 12544
NPAD = 4 * CH


def _make_seg_chunked(n_edges_padded, width):
    n_acc = CH + 8

    @functools.partial(
        pl.kernel,
        out_type=jax.ShapeDtypeStruct((NPAD, width), FP),
        mesh=plsc.VectorSubcoreMesh(core_axis_name="c", subcore_axis_name="s",
                                    num_cores=NC, num_subcores=NS),
        scratch_types=[
            pltpu.VMEM((EB,), jnp.int32),
            pltpu.VMEM((EB,), jnp.int32),
            pltpu.VMEM((EB,), jnp.int32),
            pltpu.VMEM((EB, width), FP),
            pltpu.VMEM_SHARED((n_acc, width), FP),
            pltpu.SemaphoreType.DMA,
        ],
        name=f"sc_seg_chunked_{n_edges_padded}_{width}",
    )
    def kern(table, src, dst, msum, idx_s, idx_d, adj, rows, acc, sem):
        cid = lax.axis_index("c")
        sid = lax.axis_index("s")

        for cc in range(2):
            base = (cid * 2 + cc) * CH

            _fill(rows, EB, width, 0.0)

            def zero_blk(off, sz):
                pltpu.sync_copy(rows.at[pl.ds(0, sz)], acc.at[pl.ds(off, sz)])

            _round_robin(n_acc, EB, NS, sid, zero_blk)
            plsc.subcore_barrier()

            def edge_blk(off, sz):
                pltpu.sync_copy(src.at[pl.ds(off, EB)], idx_s)
                pltpu.sync_copy(dst.at[pl.ds(off, EB)], idx_d)
                for j in range(EB // LN):
                    d = idx_d[pl.ds(j * LN, LN)]
                    loc = d - base
                    ok = (loc >= 0) & (loc < CH)
                    adj[pl.ds(j * LN, LN)] = jnp.where(ok, loc, CH)
                pltpu.async_copy(table.at[idx_s], rows, sem).wait()
                pltpu.sync_copy(rows, acc.at[adj], add=True)

            _round_robin(n_edges_padded, EB, NS, sid, edge_blk)
            plsc.subcore_barrier()

            def flush_blk(off, sz):
                pltpu.sync_copy(acc.at[pl.ds(off, sz)],
                                msum.at[pl.ds(base + off, sz)])

            _round_robin(CH, EB, NS, sid, flush_blk)
            plsc.subcore_barrier()

    return kern


def _dot(a, b):
    return jnp.dot(a, b, preferred_element_type=FP)


def _fg_packet(freq, flit, W_freq, b_freq, W_flit, b_flit, W_fh, b_fh):
    n = freq.shape[0]
    bn = 2000
    grid = n // bn

    def body(freq_r, flit_r, wfr, bfr, wfl, bfl, wfh, bfh, out_r):
        x1 = jax.nn.relu(freq_r[...] * wfr[...] + bfr[...])
        x2 = jax.nn.relu(_dot(flit_r[...], wfl[...]) + bfl[...])
        w = wfh[...]
        out_r[...] = jax.nn.relu(_dot(x1, w[:H]) + _dot(x2, w[H:]) + bfh[...])

    full = lambda a: pl.BlockSpec(a.shape, lambda i: (0,) * a.ndim)
    return pl.pallas_call(
        body,
        grid=(grid,),
        in_specs=[
            pl.BlockSpec((bn, 1), lambda i: (i, 0)),
            pl.BlockSpec((bn, 32), lambda i: (i, 0)),
            full(W_freq), full(b_freq), full(W_flit), full(b_flit),
            full(W_fh), full(b_fh),
        ],
        out_specs=pl.BlockSpec((bn, H), lambda i: (i, 0)),
        out_shape=jax.ShapeDtypeStruct((n, H), FP),
    )(freq, flit, W_freq, b_freq, W_flit, b_flit, W_fh, b_fh)


def _fg_router(op_type, W_op, b_op, W_fn, b_fn):
    n = op_type.shape[0]
    bn = 2000
    grid = n // bn

    def body(op_r, wop, bop, wfn, bfn, out_r):
        x = jax.nn.relu(_dot(op_r[...], wop[...]) + bop[...])
        out_r[...] = jax.nn.relu(_dot(x, wfn[...]) + bfn[...])

    full = lambda a: pl.BlockSpec(a.shape, lambda i: (0,) * a.ndim)
    return pl.pallas_call(
        body,
        grid=(grid,),
        in_specs=[
            pl.BlockSpec((bn, 4), lambda i: (i, 0)),
            full(W_op), full(b_op), full(W_fn), full(b_fn),
        ],
        out_specs=pl.BlockSpec((bn, H), lambda i: (i, 0)),
        out_shape=jax.ShapeDtypeStruct((n, H), FP),
    )(op_type, W_op, b_op, W_fn, b_fn)


def _router_update(router, h1p, h2p, Wr, br):
    n = router.shape[0]
    bn = 2000
    grid = n // bn

    def body(r_r, h1_r, h2_r, wr, br_r, out_r):
        h1 = h1_r[0] + h1_r[1]
        h2 = h2_r[0] + h2_r[1]
        w = wr[...]
        out_r[...] = r_r[...] + jax.nn.relu(
            _dot(h1, w[:H]) + _dot(h2, w[H:]) + br_r[...])

    full = lambda a: pl.BlockSpec(a.shape, lambda i: (0,) * a.ndim)
    return pl.pallas_call(
        body,
        grid=(grid,),
        in_specs=[
            pl.BlockSpec((bn, H), lambda i: (i, 0)),
            pl.BlockSpec((NC, bn, H), lambda i: (0, i, 0)),
            pl.BlockSpec((NC, bn, H), lambda i: (0, i, 0)),
            full(Wr), full(br),
        ],
        out_specs=pl.BlockSpec((bn, H), lambda i: (i, 0)),
        out_shape=jax.ShapeDtypeStruct((n, H), FP),
    )(router, h1p, h2p, Wr, br)


def _packet_update(packet, msum, cnt, Wp, bp):
    n = packet.shape[0]
    bn = 2000
    grid = n // bn

    def body(p_r, ms_r, cnt_r, wp, bp_r, out_r):
        c = jnp.maximum(cnt_r[:, :1], 1.0)
        h = ms_r[...] / c
        out_r[...] = p_r[...] + jax.nn.relu(_dot(h, wp[...]) + bp_r[...])

    full = lambda a: pl.BlockSpec(a.shape, lambda i: (0,) * a.ndim)
    return pl.pallas_call(
        body,
        grid=(grid,),
        in_specs=[
            pl.BlockSpec((bn, H), lambda i: (i, 0)),
            pl.BlockSpec((bn, H), lambda i: (i, 0)),
            pl.BlockSpec((bn, LN), lambda i: (i, 0)),
            full(Wp), full(bp),
        ],
        out_specs=pl.BlockSpec((bn, H), lambda i: (i, 0)),
        out_shape=jax.ShapeDtypeStruct((n, H), FP),
    )(packet, msum, cnt, Wp, bp)


def _readout(router, Wh1, bh1, Wh2, bh2, Wh3, bh3):
    def body(r_r, w1, b1, w2, b2, w3, b3, out_r):
        r = r_r[...]
        s = jnp.sum(r, axis=0, keepdims=True)
        m = jnp.max(r, axis=0, keepdims=True)
        w = w1[...]
        h = jax.nn.relu(_dot(s, w[:H]) + _dot(m, w[H:]) + b1[...])
        h = jax.nn.relu(_dot(h, w2[...]) + b2[...])
        out_r[...] = _dot(h, w3[...]) + b3[...]

    full = lambda a: pl.BlockSpec(a.shape, lambda i: (0,) * a.ndim)
    return pl.pallas_call(
        body,
        grid=(1,),
        in_specs=[full(router), full(Wh1), full(bh1), full(Wh2), full(bh2),
                  full(Wh3), full(bh3)],
        out_specs=pl.BlockSpec((1, 4), lambda i: (0, 0)),
        out_shape=jax.ShapeDtypeStruct((1, 4), FP),
    )(router, Wh1, bh1, Wh2, bh2, Wh3, bh3)


def _pad_edges(src, dst, pad_dst_value):
    e = src.shape[0]
    ep = ((e + EB - 1) // EB) * EB
    if ep != e:
        src = jnp.concatenate(
            [src, jnp.zeros((ep - e,), jnp.int32)])
        dst = jnp.concatenate(
            [dst, jnp.full((ep - e,), pad_dst_value, jnp.int32)])
    return src, dst, ep


def kernel(freq, flit, op_type, W_freq, b_freq, W_flit, b_flit, W_op, b_op,
           W_fh, b_fh, W_fn, b_fn, Wr1, br1, Wp1, bp1, Wr2, br2, Wp2, bp2,
           Wh1, bh1, Wh2, bh2, Wh3, bh3,
           pass_src, pass_dst, transfer_src, transfer_dst,
           connect_src, connect_dst):
    np_, nr = freq.shape[0], op_type.shape[0]

    i32 = lambda a: a.astype(jnp.int32)
    ps, pd, ep_p = _pad_edges(i32(pass_src), i32(pass_dst), nr)
    cs, cd, ep_c = _pad_edges(i32(connect_src), i32(connect_dst), nr)
    ts, td, ep_t = _pad_edges(i32(transfer_src), i32(transfer_dst), np_)

    row = lambda b: b.reshape(1, -1)
    b_freq_, b_flit_, b_op_, b_fh_, b_fn_ = map(
        row, (b_freq, b_flit, b_op, b_fh, b_fn))
    br1_, bp1_, br2_, bh1_, bh2_, bh3_ = map(
        row, (br1, bp1, br2, bh1, bh2, bh3))

    seg_pass = _make_seg_partials(nr, ep_p, H)
    seg_conn = _make_seg_partials(nr, ep_c, H)
    seg_xfer = _make_seg_chunked(ep_t, H)
    seg_cnt = _make_cnt_grouped(ep_t)

    onehot = (jnp.arange(H, dtype=jnp.int32)[None, :] // LN
              == jnp.arange(8, dtype=jnp.int32)[:, None]).astype(FP)

    packet0 = _fg_packet(freq, flit, W_freq, b_freq_, W_flit, b_flit_,
                         W_fh, b_fh_)
    router0 = _fg_router(op_type, W_op, b_op_, W_fn, b_fn_)

    h1p = seg_pass(packet0, ps, pd)
    h2p = seg_conn(router0, cs, cd)
    msum = seg_xfer(router0, ts, td)
    cntp = seg_cnt(onehot, td)
    cnt16 = (cntp[0] + cntp[1]).reshape(ND8 * 8, LN)[:np_]

    router1 = _router_update(router0, h1p, h2p, Wr1, br1_)
    packet1 = _packet_update(packet0, msum[:np_], cnt16, Wp1, bp1_)

    h1p2 = seg_pass(packet1, ps, pd)
    h2p2 = seg_conn(router1, cs, cd)
    router2 = _router_update(router1, h1p2, h2p2, Wr2, br2_)

    return _readout(router2, Wh1, bh1_, Wh2, bh2_, Wh3, bh3_)


# speedup vs baseline: 3.1897x; 1.7929x over previous
import functools

import jax
import jax.numpy as jnp
from jax import lax
from jax.experimental import pallas as pl
from jax.experimental.pallas import tpu as pltpu
from jax.experimental.pallas import tpu_sc as plsc

H = 128
NC = 2
NS = 16
LN = 16
EB = 96

FP = jnp.float32


def _fill(ref, nrows, ncols, value):
    vals = jnp.full((LN,), value, FP)

    def body(i, c):
        for j in range(ncols // LN):
            ref[i, pl.ds(j * LN, LN)] = vals
        return c

    lax.fori_loop(0, nrows, body, 0)


def _round_robin(total, bs, nworkers, wid, fn):
    nf = total // bs
    rem = total - nf * bs
    kmax = (nf + nworkers - 1) // nworkers
    if kmax > 0:
        def body(k, c):
            b = wid + k * nworkers

            @pl.when(b < nf)
            def _():
                fn(b * bs, bs)

            return c

        lax.fori_loop(0, kmax, body, 0)
    if rem:
        @pl.when(wid == 0)
        def _():
            fn(nf * bs, rem)


def _make_seg_partials(n_dst, n_edges_padded, width):
    n_acc = n_dst + 8

    @functools.partial(
        pl.kernel,
        out_type=jax.ShapeDtypeStruct((NC, n_dst, width), FP),
        mesh=plsc.VectorSubcoreMesh(core_axis_name="c", subcore_axis_name="s",
                                    num_cores=NC, num_subcores=NS),
        scratch_types=[
            pltpu.VMEM((EB,), jnp.int32),
            pltpu.VMEM((EB,), jnp.int32),
            pltpu.VMEM((EB, width), FP),
            pltpu.VMEM_SHARED((n_acc, width), FP),
            pltpu.SemaphoreType.DMA,
        ],
        name=f"sc_seg_partials_{n_dst}_{n_edges_padded}_{width}",
    )
    def kern(table, src, dst, out, idx_s, idx_d, rows, acc, sem):
        cid = lax.axis_index("c")
        sid = lax.axis_index("s")
        wid = cid * NS + sid

        _fill(rows, EB, width, 0.0)

        def zero_blk(off, sz):
            pltpu.sync_copy(rows.at[pl.ds(0, sz)], acc.at[pl.ds(off, sz)])

        _round_robin(n_acc, EB, NS, sid, zero_blk)
        plsc.subcore_barrier()

        def edge_blk(off, sz):
            pltpu.sync_copy(src.at[pl.ds(off, EB)], idx_s)
            pltpu.sync_copy(dst.at[pl.ds(off, EB)], idx_d)
            pltpu.async_copy(table.at[idx_s], rows, sem).wait()
            pltpu.sync_copy(rows, acc.at[idx_d], add=True)

        _round_robin(n_edges_padded, EB, NC * NS, wid, edge_blk)
        plsc.subcore_barrier()

        def flush_blk(off, sz):
            pltpu.sync_copy(acc.at[pl.ds(off, sz)], out.at[cid, pl.ds(off, sz)])

        _round_robin(n_dst, EB, NS, sid, flush_blk)

    return kern


HR = 480


def _make_cnt_hist(n_edges_padded):
    @functools.partial(
        pl.kernel,
        out_type=jax.ShapeDtypeStruct((NC, NS, HR * H), FP),
        mesh=plsc.VectorSubcoreMesh(core_axis_name="c", subcore_axis_name="s",
                                    num_cores=NC, num_subcores=NS),
        scratch_types=[
            pltpu.VMEM((EB,), jnp.int32),
            pltpu.VMEM((HR * H,), FP),
        ],
        compiler_params=pltpu.CompilerParams(needs_layout_passes=False),
        name=f"sc_cnt_hist_{n_edges_padded}",
    )
    def kern(dst, out, idx_d, hist):
        cid = lax.axis_index("c")
        sid = lax.axis_index("s")
        wid = cid * NS + sid

        zeros = jnp.zeros((LN,), FP)

        def zfill(i, c):
            hist[pl.ds(i * LN, LN)] = zeros
            return c

        lax.fori_loop(0, HR * H // LN, zfill, 0)

        ones = jnp.full((LN,), 1.0, FP)

        def edge_blk(off, sz):
            pltpu.sync_copy(dst.at[pl.ds(off, EB)], idx_d)
            for j in range(EB // LN):
                d = idx_d[pl.ds(j * LN, LN)]
                plsc.addupdate_scatter(hist, [d], ones)

        _round_robin(n_edges_padded, EB, NC * NS, wid, edge_blk)

        pltpu.sync_copy(hist, out.at[cid, sid])

    return kern


CH = 12544
NPAD = 4 * CH


def _make_seg_chunked(n_edges_padded, width):
    n_acc = CH + 8

    @functools.partial(
        pl.kernel,
        out_type=jax.ShapeDtypeStruct((NPAD, width), FP),
        mesh=plsc.VectorSubcoreMesh(core_axis_name="c", subcore_axis_name="s",
                                    num_cores=NC, num_subcores=NS),
        scratch_types=[
            pltpu.VMEM((EB,), jnp.int32),
            pltpu.VMEM((EB,), jnp.int32),
            pltpu.VMEM((EB,), jnp.int32),
            pltpu.VMEM((EB, width), FP),
            pltpu.VMEM_SHARED((n_acc, width), FP),
            pltpu.SemaphoreType.DMA,
        ],
        name=f"sc_seg_chunked_{n_edges_padded}_{width}",
    )
    def kern(table, src, dst, msum, idx_s, idx_d, adj, rows, acc, sem):
        cid = lax.axis_index("c")
        sid = lax.axis_index("s")

        for cc in range(2):
            base = (cid * 2 + cc) * CH

            _fill(rows, EB, width, 0.0)

            def zero_blk(off, sz):
                pltpu.sync_copy(rows.at[pl.ds(0, sz)], acc.at[pl.ds(off, sz)])

            _round_robin(n_acc, EB, NS, sid, zero_blk)
            plsc.subcore_barrier()

            def edge_blk(off, sz):
                pltpu.sync_copy(src.at[pl.ds(off, EB)], idx_s)
                pltpu.sync_copy(dst.at[pl.ds(off, EB)], idx_d)
                for j in range(EB // LN):
                    d = idx_d[pl.ds(j * LN, LN)]
                    loc = d - base
                    ok = (loc >= 0) & (loc < CH)
                    adj[pl.ds(j * LN, LN)] = jnp.where(ok, loc, CH)
                pltpu.async_copy(table.at[idx_s], rows, sem).wait()
                pltpu.sync_copy(rows, acc.at[adj], add=True)

            _round_robin(n_edges_padded, EB, NS, sid, edge_blk)
            plsc.subcore_barrier()

            def flush_blk(off, sz):
                pltpu.sync_copy(acc.at[pl.ds(off, sz)],
                                msum.at[pl.ds(base + off, sz)])

            _round_robin(CH, EB, NS, sid, flush_blk)
            plsc.subcore_barrier()

    return kern


def _dot(a, b):
    return jnp.dot(a, b, preferred_element_type=FP)


def _fg_packet(freq, flit, W_freq, b_freq, W_flit, b_flit, W_fh, b_fh):
    n = freq.shape[0]
    bn = 2000
    grid = n // bn

    def body(freq_r, flit_r, wfr, bfr, wfl, bfl, wfh, bfh, out_r):
        x1 = jax.nn.relu(freq_r[...] * wfr[...] + bfr[...])
        x2 = jax.nn.relu(_dot(flit_r[...], wfl[...]) + bfl[...])
        w = wfh[...]
        out_r[...] = jax.nn.relu(_dot(x1, w[:H]) + _dot(x2, w[H:]) + bfh[...])

    full = lambda a: pl.BlockSpec(a.shape, lambda i: (0,) * a.ndim)
    return pl.pallas_call(
        body,
        grid=(grid,),
        in_specs=[
            pl.BlockSpec((bn, 1), lambda i: (i, 0)),
            pl.BlockSpec((bn, 32), lambda i: (i, 0)),
            full(W_freq), full(b_freq), full(W_flit), full(b_flit),
            full(W_fh), full(b_fh),
        ],
        out_specs=pl.BlockSpec((bn, H), lambda i: (i, 0)),
        out_shape=jax.ShapeDtypeStruct((n, H), FP),
    )(freq, flit, W_freq, b_freq, W_flit, b_flit, W_fh, b_fh)


def _fg_router(op_type, W_op, b_op, W_fn, b_fn):
    n = op_type.shape[0]
    bn = 2000
    grid = n // bn

    def body(op_r, wop, bop, wfn, bfn, out_r):
        x = jax.nn.relu(_dot(op_r[...], wop[...]) + bop[...])
        out_r[...] = jax.nn.relu(_dot(x, wfn[...]) + bfn[...])

    full = lambda a: pl.BlockSpec(a.shape, lambda i: (0,) * a.ndim)
    return pl.pallas_call(
        body,
        grid=(grid,),
        in_specs=[
            pl.BlockSpec((bn, 4), lambda i: (i, 0)),
            full(W_op), full(b_op), full(W_fn), full(b_fn),
        ],
        out_specs=pl.BlockSpec((bn, H), lambda i: (i, 0)),
        out_shape=jax.ShapeDtypeStruct((n, H), FP),
    )(op_type, W_op, b_op, W_fn, b_fn)


def _router_update(router, h1p, h2p, Wr, br):
    n = router.shape[0]
    bn = 2000
    grid = n // bn

    def body(r_r, h1_r, h2_r, wr, br_r, out_r):
        h1 = h1_r[0] + h1_r[1]
        h2 = h2_r[0] + h2_r[1]
        w = wr[...]
        out_r[...] = r_r[...] + jax.nn.relu(
            _dot(h1, w[:H]) + _dot(h2, w[H:]) + br_r[...])

    full = lambda a: pl.BlockSpec(a.shape, lambda i: (0,) * a.ndim)
    return pl.pallas_call(
        body,
        grid=(grid,),
        in_specs=[
            pl.BlockSpec((bn, H), lambda i: (i, 0)),
            pl.BlockSpec((NC, bn, H), lambda i: (0, i, 0)),
            pl.BlockSpec((NC, bn, H), lambda i: (0, i, 0)),
            full(Wr), full(br),
        ],
        out_specs=pl.BlockSpec((bn, H), lambda i: (i, 0)),
        out_shape=jax.ShapeDtypeStruct((n, H), FP),
    )(router, h1p, h2p, Wr, br)


def _packet_update(packet, msum, cnt, Wp, bp):
    n = packet.shape[0]
    bn = 2000
    grid = n // bn

    def body(p_r, ms_r, cnt_r, wp, bp_r, out_r):
        c = jnp.maximum(cnt_r[:, :1], 1.0)
        h = ms_r[...] / c
        out_r[...] = p_r[...] + jax.nn.relu(_dot(h, wp[...]) + bp_r[...])

    full = lambda a: pl.BlockSpec(a.shape, lambda i: (0,) * a.ndim)
    return pl.pallas_call(
        body,
        grid=(grid,),
        in_specs=[
            pl.BlockSpec((bn, H), lambda i: (i, 0)),
            pl.BlockSpec((bn, H), lambda i: (i, 0)),
            pl.BlockSpec((bn, LN), lambda i: (i, 0)),
            full(Wp), full(bp),
        ],
        out_specs=pl.BlockSpec((bn, H), lambda i: (i, 0)),
        out_shape=jax.ShapeDtypeStruct((n, H), FP),
    )(packet, msum, cnt, Wp, bp)


def _readout(router, Wh1, bh1, Wh2, bh2, Wh3, bh3):
    def body(r_r, w1, b1, w2, b2, w3, b3, out_r):
        r = r_r[...]
        s = jnp.sum(r, axis=0, keepdims=True)
        m = jnp.max(r, axis=0, keepdims=True)
        w = w1[...]
        h = jax.nn.relu(_dot(s, w[:H]) + _dot(m, w[H:]) + b1[...])
        h = jax.nn.relu(_dot(h, w2[...]) + b2[...])
        out_r[...] = _dot(h, w3[...]) + b3[...]

    full = lambda a: pl.BlockSpec(a.shape, lambda i: (0,) * a.ndim)
    return pl.pallas_call(
        body,
        grid=(1,),
        in_specs=[full(router), full(Wh1), full(bh1), full(Wh2), full(bh2),
                  full(Wh3), full(bh3)],
        out_specs=pl.BlockSpec((1, 4), lambda i: (0, 0)),
        out_shape=jax.ShapeDtypeStruct((1, 4), FP),
    )(router, Wh1, bh1, Wh2, bh2, Wh3, bh3)


def _pad_edges(src, dst, pad_dst_value):
    e = src.shape[0]
    ep = ((e + EB - 1) // EB) * EB
    if ep != e:
        src = jnp.concatenate(
            [src, jnp.zeros((ep - e,), jnp.int32)])
        dst = jnp.concatenate(
            [dst, jnp.full((ep - e,), pad_dst_value, jnp.int32)])
    return src, dst, ep


def kernel(freq, flit, op_type, W_freq, b_freq, W_flit, b_flit, W_op, b_op,
           W_fh, b_fh, W_fn, b_fn, Wr1, br1, Wp1, bp1, Wr2, br2, Wp2, bp2,
           Wh1, bh1, Wh2, bh2, Wh3, bh3,
           pass_src, pass_dst, transfer_src, transfer_dst,
           connect_src, connect_dst):
    np_, nr = freq.shape[0], op_type.shape[0]

    i32 = lambda a: a.astype(jnp.int32)
    ps, pd, ep_p = _pad_edges(i32(pass_src), i32(pass_dst), nr)
    cs, cd, ep_c = _pad_edges(i32(connect_src), i32(connect_dst), nr)
    ts, td, ep_t = _pad_edges(i32(transfer_src), i32(transfer_dst), np_)

    row = lambda b: b.reshape(1, -1)
    b_freq_, b_flit_, b_op_, b_fh_, b_fn_ = map(
        row, (b_freq, b_flit, b_op, b_fh, b_fn))
    br1_, bp1_, br2_, bh1_, bh2_, bh3_ = map(
        row, (br1, bp1, br2, bh1, bh2, bh3))

    seg_pass = _make_seg_partials(nr, ep_p, H)
    seg_conn = _make_seg_partials(nr, ep_c, H)
    seg_xfer = _make_seg_chunked(ep_t, H)
    seg_cnt = _make_cnt_hist(ep_t)

    packet0 = _fg_packet(freq, flit, W_freq, b_freq_, W_flit, b_flit_,
                         W_fh, b_fh_)
    router0 = _fg_router(op_type, W_op, b_op_, W_fn, b_fn_)

    h1p = seg_pass(packet0, ps, pd)
    h2p = seg_conn(router0, cs, cd)
    msum = seg_xfer(router0, ts, td)
    cntp = seg_cnt(td)
    cnt1 = cntp.reshape(NC * NS, HR * H).sum(axis=0)[:np_]
    cnt16 = jnp.broadcast_to(cnt1[:, None], (np_, LN))

    router1 = _router_update(router0, h1p, h2p, Wr1, br1_)
    packet1 = _packet_update(packet0, msum[:np_], cnt16, Wp1, bp1_)

    h1p2 = seg_pass(packet1, ps, pd)
    h2p2 = seg_conn(router1, cs, cd)
    router2 = _router_update(router1, h1p2, h2p2, Wr2, br2_)

    return _readout(router2, Wh1, bh1_, Wh2, bh2_, Wh3, bh3_)


# speedup vs baseline: 4.9800x; 1.5613x over previous
import functools

import jax
import jax.numpy as jnp
from jax import lax
from jax.experimental import pallas as pl
from jax.experimental.pallas import tpu as pltpu
from jax.experimental.pallas import tpu_sc as plsc

H = 128
NC = 2
NS = 16
LN = 16
EB = 96

FP = jnp.float32


def _fill(ref, nrows, ncols, value):
    vals = jnp.full((LN,), value, FP)

    def body(i, c):
        for j in range(ncols // LN):
            ref[i, pl.ds(j * LN, LN)] = vals
        return c

    lax.fori_loop(0, nrows, body, 0)


def _round_robin(total, bs, nworkers, wid, fn):
    nf = total // bs
    rem = total - nf * bs
    kmax = (nf + nworkers - 1) // nworkers
    if kmax > 0:
        def body(k, c):
            b = wid + k * nworkers

            @pl.when(b < nf)
            def _():
                fn(b * bs, bs)

            return c

        lax.fori_loop(0, kmax, body, 0)
    if rem:
        @pl.when(wid == 0)
        def _():
            fn(nf * bs, rem)


def _edge_pipe(nf, nworkers, wid, fetch, consume):
    kmax = (nf + nworkers - 1) // nworkers
    if kmax == 0:
        return

    @pl.when(wid < nf)
    def _():
        fetch(wid, 0)

    def body(k2, c):
        for p in range(2):
            b = wid + (2 * k2 + p) * nworkers
            nb = b + nworkers

            @pl.when(nb < nf)
            def _():
                fetch(nb, 1 - p)

            @pl.when(b < nf)
            def _():
                consume(b, p)

        return c

    lax.fori_loop(0, (kmax + 1) // 2, body, 0)


def _make_seg_partials(n_dst, n_edges_padded, width):
    n_acc = n_dst + 8
    nf = n_edges_padded // EB

    @functools.partial(
        pl.kernel,
        out_type=jax.ShapeDtypeStruct((NC, n_dst, width), FP),
        mesh=plsc.VectorSubcoreMesh(core_axis_name="c", subcore_axis_name="s",
                                    num_cores=NC, num_subcores=NS),
        scratch_types=[
            pltpu.VMEM((2, EB), jnp.int32),
            pltpu.VMEM((2, EB), jnp.int32),
            pltpu.VMEM((2, EB, width), FP),
            pltpu.VMEM_SHARED((n_acc, width), FP),
            pltpu.SemaphoreType.DMA,
            pltpu.SemaphoreType.DMA,
        ],
        name=f"sc_seg_partials_{n_dst}_{n_edges_padded}_{width}",
    )
    def kern(table, src, dst, out, idx_s, idx_d, rows, acc, sem0, sem1):
        cid = lax.axis_index("c")
        sid = lax.axis_index("s")
        wid = cid * NS + sid
        sems = (sem0, sem1)

        _fill(rows.at[0], EB, width, 0.0)

        def zero_blk(off, sz):
            pltpu.sync_copy(rows.at[0, pl.ds(0, sz)], acc.at[pl.ds(off, sz)])

        _round_robin(n_acc, EB, NS, sid, zero_blk)
        plsc.subcore_barrier()

        def fetch(b, p):
            off = b * EB
            pltpu.sync_copy(src.at[pl.ds(off, EB)], idx_s.at[p])
            pltpu.sync_copy(dst.at[pl.ds(off, EB)], idx_d.at[p])
            pltpu.async_copy(table.at[idx_s.at[p]], rows.at[p], sems[p])

        def consume(b, p):
            pltpu.make_async_copy(table.at[idx_s.at[p]], rows.at[p],
                                  sems[p]).wait()
            pltpu.sync_copy(rows.at[p], acc.at[idx_d.at[p]], add=True)

        _edge_pipe(nf, NC * NS, wid, fetch, consume)
        plsc.subcore_barrier()

        def flush_blk(off, sz):
            pltpu.sync_copy(acc.at[pl.ds(off, sz)], out.at[cid, pl.ds(off, sz)])

        _round_robin(n_dst, EB, NS, sid, flush_blk)

    return kern


HR = 480


def _make_cnt_hist(n_edges_padded):
    @functools.partial(
        pl.kernel,
        out_type=jax.ShapeDtypeStruct((NC, NS, HR * H), FP),
        mesh=plsc.VectorSubcoreMesh(core_axis_name="c", subcore_axis_name="s",
                                    num_cores=NC, num_subcores=NS),
        scratch_types=[
            pltpu.VMEM((EB,), jnp.int32),
            pltpu.VMEM((HR * H,), FP),
        ],
        compiler_params=pltpu.CompilerParams(needs_layout_passes=False),
        name=f"sc_cnt_hist_{n_edges_padded}",
    )
    def kern(dst, out, idx_d, hist):
        cid = lax.axis_index("c")
        sid = lax.axis_index("s")
        wid = cid * NS + sid

        zeros = jnp.zeros((LN,), FP)

        def zfill(i, c):
            hist[pl.ds(i * LN, LN)] = zeros
            return c

        lax.fori_loop(0, HR * H // LN, zfill, 0)

        ones = jnp.full((LN,), 1.0, FP)

        def edge_blk(off, sz):
            pltpu.sync_copy(dst.at[pl.ds(off, EB)], idx_d)
            for j in range(EB // LN):
                d = idx_d[pl.ds(j * LN, LN)]
                plsc.addupdate_scatter(hist, [d], ones)

        _round_robin(n_edges_padded, EB, NC * NS, wid, edge_blk)

        pltpu.sync_copy(hist, out.at[cid, sid])

    return kern


CH = 12544
NPAD = 4 * CH


def _make_seg_chunked(n_edges_padded, width):
    n_acc = CH + 8
    nf = n_edges_padded // EB

    @functools.partial(
        pl.kernel,
        out_type=jax.ShapeDtypeStruct((NPAD, width), FP),
        mesh=plsc.VectorSubcoreMesh(core_axis_name="c", subcore_axis_name="s",
                                    num_cores=NC, num_subcores=NS),
        scratch_types=[
            pltpu.VMEM((2, EB), jnp.int32),
            pltpu.VMEM((2, EB), jnp.int32),
            pltpu.VMEM((2, EB), jnp.int32),
            pltpu.VMEM((2, EB, width), FP),
            pltpu.VMEM_SHARED((n_acc, width), FP),
            pltpu.SemaphoreType.DMA,
            pltpu.SemaphoreType.DMA,
        ],
        name=f"sc_seg_chunked_{n_edges_padded}_{width}",
    )
    def kern(table, src, dst, msum, idx_s, idx_d, adj, rows, acc, sem0, sem1):
        cid = lax.axis_index("c")
        sid = lax.axis_index("s")
        sems = (sem0, sem1)

        for cc in range(2):
            base = (cid * 2 + cc) * CH

            _fill(rows.at[0], EB, width, 0.0)

            def zero_blk(off, sz):
                pltpu.sync_copy(rows.at[0, pl.ds(0, sz)],
                                acc.at[pl.ds(off, sz)])

            _round_robin(n_acc, EB, NS, sid, zero_blk)
            plsc.subcore_barrier()

            def fetch(b, p):
                off = b * EB
                pltpu.sync_copy(src.at[pl.ds(off, EB)], idx_s.at[p])
                pltpu.sync_copy(dst.at[pl.ds(off, EB)], idx_d.at[p])
                pltpu.async_copy(table.at[idx_s.at[p]], rows.at[p], sems[p])

            def consume(b, p):
                for j in range(EB // LN):
                    d = idx_d[p, pl.ds(j * LN, LN)]
                    loc = d - base
                    ok = (loc >= 0) & (loc < CH)
                    adj[p, pl.ds(j * LN, LN)] = jnp.where(ok, loc, CH)
                pltpu.make_async_copy(table.at[idx_s.at[p]], rows.at[p],
                                      sems[p]).wait()
                pltpu.sync_copy(rows.at[p], acc.at[adj.at[p]], add=True)

            _edge_pipe(nf, NS, sid, fetch, consume)
            plsc.subcore_barrier()

            def flush_blk(off, sz):
                pltpu.sync_copy(acc.at[pl.ds(off, sz)],
                                msum.at[pl.ds(base + off, sz)])

            _round_robin(CH, EB, NS, sid, flush_blk)
            plsc.subcore_barrier()

    return kern


def _dot(a, b):
    return jnp.dot(a, b, preferred_element_type=FP)


def _fg_packet(freq, flit, W_freq, b_freq, W_flit, b_flit, W_fh, b_fh):
    n = freq.shape[0]
    bn = 2000
    grid = n // bn

    def body(freq_r, flit_r, wfr, bfr, wfl, bfl, wfh, bfh, out_r):
        x1 = jax.nn.relu(freq_r[...] * wfr[...] + bfr[...])
        x2 = jax.nn.relu(_dot(flit_r[...], wfl[...]) + bfl[...])
        w = wfh[...]
        out_r[...] = jax.nn.relu(_dot(x1, w[:H]) + _dot(x2, w[H:]) + bfh[...])

    full = lambda a: pl.BlockSpec(a.shape, lambda i: (0,) * a.ndim)
    return pl.pallas_call(
        body,
        grid=(grid,),
        in_specs=[
            pl.BlockSpec((bn, 1), lambda i: (i, 0)),
            pl.BlockSpec((bn, 32), lambda i: (i, 0)),
            full(W_freq), full(b_freq), full(W_flit), full(b_flit),
            full(W_fh), full(b_fh),
        ],
        out_specs=pl.BlockSpec((bn, H), lambda i: (i, 0)),
        out_shape=jax.ShapeDtypeStruct((n, H), FP),
    )(freq, flit, W_freq, b_freq, W_flit, b_flit, W_fh, b_fh)


def _fg_router(op_type, W_op, b_op, W_fn, b_fn):
    n = op_type.shape[0]
    bn = 2000
    grid = n // bn

    def body(op_r, wop, bop, wfn, bfn, out_r):
        x = jax.nn.relu(_dot(op_r[...], wop[...]) + bop[...])
        out_r[...] = jax.nn.relu(_dot(x, wfn[...]) + bfn[...])

    full = lambda a: pl.BlockSpec(a.shape, lambda i: (0,) * a.ndim)
    return pl.pallas_call(
        body,
        grid=(grid,),
        in_specs=[
            pl.BlockSpec((bn, 4), lambda i: (i, 0)),
            full(W_op), full(b_op), full(W_fn), full(b_fn),
        ],
        out_specs=pl.BlockSpec((bn, H), lambda i: (i, 0)),
        out_shape=jax.ShapeDtypeStruct((n, H), FP),
    )(op_type, W_op, b_op, W_fn, b_fn)


def _router_update(router, h1p, h2p, Wr, br):
    n = router.shape[0]
    bn = 2000
    grid = n // bn

    def body(r_r, h1_r, h2_r, wr, br_r, out_r):
        h1 = h1_r[0] + h1_r[1]
        h2 = h2_r[0] + h2_r[1]
        w = wr[...]
        out_r[...] = r_r[...] + jax.nn.relu(
            _dot(h1, w[:H]) + _dot(h2, w[H:]) + br_r[...])

    full = lambda a: pl.BlockSpec(a.shape, lambda i: (0,) * a.ndim)
    return pl.pallas_call(
        body,
        grid=(grid,),
        in_specs=[
            pl.BlockSpec((bn, H), lambda i: (i, 0)),
            pl.BlockSpec((NC, bn, H), lambda i: (0, i, 0)),
            pl.BlockSpec((NC, bn, H), lambda i: (0, i, 0)),
            full(Wr), full(br),
        ],
        out_specs=pl.BlockSpec((bn, H), lambda i: (i, 0)),
        out_shape=jax.ShapeDtypeStruct((n, H), FP),
    )(router, h1p, h2p, Wr, br)


def _packet_update(packet, msum, cnt, Wp, bp):
    n = packet.shape[0]
    bn = 2000
    grid = n // bn

    def body(p_r, ms_r, cnt_r, wp, bp_r, out_r):
        c = jnp.maximum(cnt_r[:, :1], 1.0)
        h = ms_r[...] / c
        out_r[...] = p_r[...] + jax.nn.relu(_dot(h, wp[...]) + bp_r[...])

    full = lambda a: pl.BlockSpec(a.shape, lambda i: (0,) * a.ndim)
    return pl.pallas_call(
        body,
        grid=(grid,),
        in_specs=[
            pl.BlockSpec((bn, H), lambda i: (i, 0)),
            pl.BlockSpec((bn, H), lambda i: (i, 0)),
            pl.BlockSpec((bn, LN), lambda i: (i, 0)),
            full(Wp), full(bp),
        ],
        out_specs=pl.BlockSpec((bn, H), lambda i: (i, 0)),
        out_shape=jax.ShapeDtypeStruct((n, H), FP),
    )(packet, msum, cnt, Wp, bp)


def _readout(router, Wh1, bh1, Wh2, bh2, Wh3, bh3):
    def body(r_r, w1, b1, w2, b2, w3, b3, out_r):
        r = r_r[...]
        s = jnp.sum(r, axis=0, keepdims=True)
        m = jnp.max(r, axis=0, keepdims=True)
        w = w1[...]
        h = jax.nn.relu(_dot(s, w[:H]) + _dot(m, w[H:]) + b1[...])
        h = jax.nn.relu(_dot(h, w2[...]) + b2[...])
        out_r[...] = _dot(h, w3[...]) + b3[...]

    full = lambda a: pl.BlockSpec(a.shape, lambda i: (0,) * a.ndim)
    return pl.pallas_call(
        body,
        grid=(1,),
        in_specs=[full(router), full(Wh1), full(bh1), full(Wh2), full(bh2),
                  full(Wh3), full(bh3)],
        out_specs=pl.BlockSpec((1, 4), lambda i: (0, 0)),
        out_shape=jax.ShapeDtypeStruct((1, 4), FP),
    )(router, Wh1, bh1, Wh2, bh2, Wh3, bh3)


def _pad_edges(src, dst, pad_dst_value):
    e = src.shape[0]
    ep = ((e + EB - 1) // EB) * EB
    if ep != e:
        src = jnp.concatenate(
            [src, jnp.zeros((ep - e,), jnp.int32)])
        dst = jnp.concatenate(
            [dst, jnp.full((ep - e,), pad_dst_value, jnp.int32)])
    return src, dst, ep


def kernel(freq, flit, op_type, W_freq, b_freq, W_flit, b_flit, W_op, b_op,
           W_fh, b_fh, W_fn, b_fn, Wr1, br1, Wp1, bp1, Wr2, br2, Wp2, bp2,
           Wh1, bh1, Wh2, bh2, Wh3, bh3,
           pass_src, pass_dst, transfer_src, transfer_dst,
           connect_src, connect_dst):
    np_, nr = freq.shape[0], op_type.shape[0]

    i32 = lambda a: a.astype(jnp.int32)
    ps, pd, ep_p = _pad_edges(i32(pass_src), i32(pass_dst), nr)
    cs, cd, ep_c = _pad_edges(i32(connect_src), i32(connect_dst), nr)
    ts, td, ep_t = _pad_edges(i32(transfer_src), i32(transfer_dst), np_)

    row = lambda b: b.reshape(1, -1)
    b_freq_, b_flit_, b_op_, b_fh_, b_fn_ = map(
        row, (b_freq, b_flit, b_op, b_fh, b_fn))
    br1_, bp1_, br2_, bh1_, bh2_, bh3_ = map(
        row, (br1, bp1, br2, bh1, bh2, bh3))

    seg_pass = _make_seg_partials(nr, ep_p, H)
    seg_conn = _make_seg_partials(nr, ep_c, H)
    seg_xfer = _make_seg_chunked(ep_t, H)
    seg_cnt = _make_cnt_hist(ep_t)

    packet0 = _fg_packet(freq, flit, W_freq, b_freq_, W_flit, b_flit_,
                         W_fh, b_fh_)
    router0 = _fg_router(op_type, W_op, b_op_, W_fn, b_fn_)

    h1p = seg_pass(packet0, ps, pd)
    h2p = seg_conn(router0, cs, cd)
    msum = seg_xfer(router0, ts, td)
    cntp = seg_cnt(td)
    cnt1 = cntp.reshape(NC * NS, HR * H).sum(axis=0)[:np_]
    cnt16 = jnp.broadcast_to(cnt1[:, None], (np_, LN))

    router1 = _router_update(router0, h1p, h2p, Wr1, br1_)
    packet1 = _packet_update(packet0, msum[:np_], cnt16, Wp1, bp1_)

    h1p2 = seg_pass(packet1, ps, pd)
    h2p2 = seg_conn(router1, cs, cd)
    router2 = _router_update(router1, h1p2, h2p2, Wr2, br2_)

    return _readout(router2, Wh1, bh1_, Wh2, bh2_, Wh3, bh3_)


# speedup vs baseline: 5.0012x; 1.0043x over previous
import functools

import jax
import jax.numpy as jnp
from jax import lax
from jax.experimental import pallas as pl
from jax.experimental.pallas import tpu as pltpu
from jax.experimental.pallas import tpu_sc as plsc

H = 128
NC = 2
NS = 16
LN = 16
EB = 96

FP = jnp.float32


def _fill(ref, nrows, ncols, value):
    vals = jnp.full((LN,), value, FP)

    def body(i, c):
        for j in range(ncols // LN):
            ref[i, pl.ds(j * LN, LN)] = vals
        return c

    lax.fori_loop(0, nrows, body, 0)


def _round_robin(total, bs, nworkers, wid, fn):
    nf = total // bs
    rem = total - nf * bs
    kmax = (nf + nworkers - 1) // nworkers
    if kmax > 0:
        def body(k, c):
            b = wid + k * nworkers

            @pl.when(b < nf)
            def _():
                fn(b * bs, bs)

            return c

        lax.fori_loop(0, kmax, body, 0)
    if rem:
        @pl.when(wid == 0)
        def _():
            fn(nf * bs, rem)


def _edge_pipe(nf, nworkers, wid, fetch, consume, nbuf):
    kmax = (nf + nworkers - 1) // nworkers
    if kmax == 0:
        return

    for q in range(nbuf - 1):
        b0 = wid + q * nworkers

        @pl.when(b0 < nf)
        def _(b0=b0, q=q):
            fetch(b0, q)

    def body(k2, c):
        for p in range(nbuf):
            b = wid + (nbuf * k2 + p) * nworkers
            nb = b + (nbuf - 1) * nworkers

            @pl.when(nb < nf)
            def _():
                fetch(nb, (p + nbuf - 1) % nbuf)

            @pl.when(b < nf)
            def _():
                consume(b, p)

        return c

    lax.fori_loop(0, (kmax + nbuf - 1) // nbuf, body, 0)


def _make_seg_partials(n_dst, n_edges_padded, width):
    n_acc = n_dst + 8
    nf = n_edges_padded // EB

    @functools.partial(
        pl.kernel,
        out_type=jax.ShapeDtypeStruct((NC, n_dst, width), FP),
        mesh=plsc.VectorSubcoreMesh(core_axis_name="c", subcore_axis_name="s",
                                    num_cores=NC, num_subcores=NS),
        scratch_types=[
            pltpu.VMEM((4, EB), jnp.int32),
            pltpu.VMEM((4, EB), jnp.int32),
            pltpu.VMEM((4, EB, width), FP),
            pltpu.VMEM_SHARED((n_acc, width), FP),
        ] + [pltpu.SemaphoreType.DMA] * 4,
        name=f"sc_seg_partials_{n_dst}_{n_edges_padded}_{width}",
    )
    def kern(table, src, dst, out, idx_s, idx_d, rows, acc, *sems):
        cid = lax.axis_index("c")
        sid = lax.axis_index("s")
        wid = cid * NS + sid

        _fill(rows.at[0], EB, width, 0.0)

        def zero_blk(off, sz):
            pltpu.sync_copy(rows.at[0, pl.ds(0, sz)], acc.at[pl.ds(off, sz)])

        _round_robin(n_acc, EB, NS, sid, zero_blk)
        plsc.subcore_barrier()

        def fetch(b, p):
            off = b * EB
            pltpu.sync_copy(src.at[pl.ds(off, EB)], idx_s.at[p])
            pltpu.sync_copy(dst.at[pl.ds(off, EB)], idx_d.at[p])
            pltpu.async_copy(table.at[idx_s.at[p]], rows.at[p], sems[p])

        def consume(b, p):
            pltpu.make_async_copy(table.at[idx_s.at[p]], rows.at[p],
                                  sems[p]).wait()
            pltpu.sync_copy(rows.at[p], acc.at[idx_d.at[p]], add=True)

        _edge_pipe(nf, NC * NS, wid, fetch, consume, 4)
        plsc.subcore_barrier()

        def flush_blk(off, sz):
            pltpu.sync_copy(acc.at[pl.ds(off, sz)], out.at[cid, pl.ds(off, sz)])

        _round_robin(n_dst, EB, NS, sid, flush_blk)

    return kern


HR = 480


def _make_cnt_hist(n_edges_padded):
    @functools.partial(
        pl.kernel,
        out_type=jax.ShapeDtypeStruct((NC, NS, HR * H), FP),
        mesh=plsc.VectorSubcoreMesh(core_axis_name="c", subcore_axis_name="s",
                                    num_cores=NC, num_subcores=NS),
        scratch_types=[
            pltpu.VMEM((EB,), jnp.int32),
            pltpu.VMEM((HR * H,), FP),
        ],
        compiler_params=pltpu.CompilerParams(needs_layout_passes=False),
        name=f"sc_cnt_hist_{n_edges_padded}",
    )
    def kern(dst, out, idx_d, hist):
        cid = lax.axis_index("c")
        sid = lax.axis_index("s")
        wid = cid * NS + sid

        zeros = jnp.zeros((LN,), FP)

        def zfill(i, c):
            hist[pl.ds(i * LN, LN)] = zeros
            return c

        lax.fori_loop(0, HR * H // LN, zfill, 0)

        ones = jnp.full((LN,), 1.0, FP)

        def edge_blk(off, sz):
            pltpu.sync_copy(dst.at[pl.ds(off, EB)], idx_d)
            for j in range(EB // LN):
                d = idx_d[pl.ds(j * LN, LN)]
                plsc.addupdate_scatter(hist, [d], ones)

        _round_robin(n_edges_padded, EB, NC * NS, wid, edge_blk)

        pltpu.sync_copy(hist, out.at[cid, sid])

    return kern


CH = 12544
NPAD = 4 * CH


def _make_seg_chunked(n_edges_padded, width):
    n_acc = CH + 8
    nf = n_edges_padded // EB

    @functools.partial(
        pl.kernel,
        out_type=jax.ShapeDtypeStruct((NPAD, width), FP),
        mesh=plsc.VectorSubcoreMesh(core_axis_name="c", subcore_axis_name="s",
                                    num_cores=NC, num_subcores=NS),
        scratch_types=[
            pltpu.VMEM((2, EB), jnp.int32),
            pltpu.VMEM((2, EB), jnp.int32),
            pltpu.VMEM((2, EB), jnp.int32),
            pltpu.VMEM((2, EB, width), FP),
            pltpu.VMEM_SHARED((n_acc, width), FP),
        ] + [pltpu.SemaphoreType.DMA] * 2,
        name=f"sc_seg_chunked_{n_edges_padded}_{width}",
    )
    def kern(table, src, dst, msum, idx_s, idx_d, adj, rows, acc, *sems):
        cid = lax.axis_index("c")
        sid = lax.axis_index("s")

        for cc in range(2):
            base = (cid * 2 + cc) * CH

            _fill(rows.at[0], EB, width, 0.0)

            def zero_blk(off, sz):
                pltpu.sync_copy(rows.at[0, pl.ds(0, sz)],
                                acc.at[pl.ds(off, sz)])

            _round_robin(n_acc, EB, NS, sid, zero_blk)
            plsc.subcore_barrier()

            def fetch(b, p):
                off = b * EB
                pltpu.sync_copy(src.at[pl.ds(off, EB)], idx_s.at[p])
                pltpu.sync_copy(dst.at[pl.ds(off, EB)], idx_d.at[p])
                pltpu.async_copy(table.at[idx_s.at[p]], rows.at[p], sems[p])

            def consume(b, p):
                for j in range(EB // LN):
                    d = idx_d[p, pl.ds(j * LN, LN)]
                    loc = d - base
                    ok = (loc >= 0) & (loc < CH)
                    adj[p, pl.ds(j * LN, LN)] = jnp.where(ok, loc, CH)
                pltpu.make_async_copy(table.at[idx_s.at[p]], rows.at[p],
                                      sems[p]).wait()
                pltpu.sync_copy(rows.at[p], acc.at[adj.at[p]], add=True)

            _edge_pipe(nf, NS, sid, fetch, consume, 2)
            plsc.subcore_barrier()

            def flush_blk(off, sz):
                pltpu.sync_copy(acc.at[pl.ds(off, sz)],
                                msum.at[pl.ds(base + off, sz)])

            _round_robin(CH, EB, NS, sid, flush_blk)
            plsc.subcore_barrier()

    return kern


def _dot(a, b):
    return jnp.dot(a, b, preferred_element_type=FP)


def _fg_packet(freq, flit, W_freq, b_freq, W_flit, b_flit, W_fh, b_fh):
    n = freq.shape[0]
    bn = 2000
    grid = n // bn

    def body(freq_r, flit_r, wfr, bfr, wfl, bfl, wfh, bfh, out_r):
        x1 = jax.nn.relu(freq_r[...] * wfr[...] + bfr[...])
        x2 = jax.nn.relu(_dot(flit_r[...], wfl[...]) + bfl[...])
        w = wfh[...]
        out_r[...] = jax.nn.relu(_dot(x1, w[:H]) + _dot(x2, w[H:]) + bfh[...])

    full = lambda a: pl.BlockSpec(a.shape, lambda i: (0,) * a.ndim)
    return pl.pallas_call(
        body,
        grid=(grid,),
        in_specs=[
            pl.BlockSpec((bn, 1), lambda i: (i, 0)),
            pl.BlockSpec((bn, 32), lambda i: (i, 0)),
            full(W_freq), full(b_freq), full(W_flit), full(b_flit),
            full(W_fh), full(b_fh),
        ],
        out_specs=pl.BlockSpec((bn, H), lambda i: (i, 0)),
        out_shape=jax.ShapeDtypeStruct((n, H), FP),
    )(freq, flit, W_freq, b_freq, W_flit, b_flit, W_fh, b_fh)


def _fg_router(op_type, W_op, b_op, W_fn, b_fn):
    n = op_type.shape[0]
    bn = 2000
    grid = n // bn

    def body(op_r, wop, bop, wfn, bfn, out_r):
        x = jax.nn.relu(_dot(op_r[...], wop[...]) + bop[...])
        out_r[...] = jax.nn.relu(_dot(x, wfn[...]) + bfn[...])

    full = lambda a: pl.BlockSpec(a.shape, lambda i: (0,) * a.ndim)
    return pl.pallas_call(
        body,
        grid=(grid,),
        in_specs=[
            pl.BlockSpec((bn, 4), lambda i: (i, 0)),
            full(W_op), full(b_op), full(W_fn), full(b_fn),
        ],
        out_specs=pl.BlockSpec((bn, H), lambda i: (i, 0)),
        out_shape=jax.ShapeDtypeStruct((n, H), FP),
    )(op_type, W_op, b_op, W_fn, b_fn)


def _router_update(router, h1p, h2p, Wr, br):
    n = router.shape[0]
    bn = 2000
    grid = n // bn

    def body(r_r, h1_r, h2_r, wr, br_r, out_r):
        h1 = h1_r[0] + h1_r[1]
        h2 = h2_r[0] + h2_r[1]
        w = wr[...]
        out_r[...] = r_r[...] + jax.nn.relu(
            _dot(h1, w[:H]) + _dot(h2, w[H:]) + br_r[...])

    full = lambda a: pl.BlockSpec(a.shape, lambda i: (0,) * a.ndim)
    return pl.pallas_call(
        body,
        grid=(grid,),
        in_specs=[
            pl.BlockSpec((bn, H), lambda i: (i, 0)),
            pl.BlockSpec((NC, bn, H), lambda i: (0, i, 0)),
            pl.BlockSpec((NC, bn, H), lambda i: (0, i, 0)),
            full(Wr), full(br),
        ],
        out_specs=pl.BlockSpec((bn, H), lambda i: (i, 0)),
        out_shape=jax.ShapeDtypeStruct((n, H), FP),
    )(router, h1p, h2p, Wr, br)


def _packet_update(packet, msum, cnt, Wp, bp):
    n = packet.shape[0]
    bn = 2000
    grid = n // bn

    def body(p_r, ms_r, cnt_r, wp, bp_r, out_r):
        c = jnp.maximum(cnt_r[:, :1], 1.0)
        h = ms_r[...] / c
        out_r[...] = p_r[...] + jax.nn.relu(_dot(h, wp[...]) + bp_r[...])

    full = lambda a: pl.BlockSpec(a.shape, lambda i: (0,) * a.ndim)
    return pl.pallas_call(
        body,
        grid=(grid,),
        in_specs=[
            pl.BlockSpec((bn, H), lambda i: (i, 0)),
            pl.BlockSpec((bn, H), lambda i: (i, 0)),
            pl.BlockSpec((bn, LN), lambda i: (i, 0)),
            full(Wp), full(bp),
        ],
        out_specs=pl.BlockSpec((bn, H), lambda i: (i, 0)),
        out_shape=jax.ShapeDtypeStruct((n, H), FP),
    )(packet, msum, cnt, Wp, bp)


def _readout(router, Wh1, bh1, Wh2, bh2, Wh3, bh3):
    def body(r_r, w1, b1, w2, b2, w3, b3, out_r):
        r = r_r[...]
        s = jnp.sum(r, axis=0, keepdims=True)
        m = jnp.max(r, axis=0, keepdims=True)
        w = w1[...]
        h = jax.nn.relu(_dot(s, w[:H]) + _dot(m, w[H:]) + b1[...])
        h = jax.nn.relu(_dot(h, w2[...]) + b2[...])
        out_r[...] = _dot(h, w3[...]) + b3[...]

    full = lambda a: pl.BlockSpec(a.shape, lambda i: (0,) * a.ndim)
    return pl.pallas_call(
        body,
        grid=(1,),
        in_specs=[full(router), full(Wh1), full(bh1), full(Wh2), full(bh2),
                  full(Wh3), full(bh3)],
        out_specs=pl.BlockSpec((1, 4), lambda i: (0, 0)),
        out_shape=jax.ShapeDtypeStruct((1, 4), FP),
    )(router, Wh1, bh1, Wh2, bh2, Wh3, bh3)


def _pad_edges(src, dst, pad_dst_value):
    e = src.shape[0]
    ep = ((e + EB - 1) // EB) * EB
    if ep != e:
        src = jnp.concatenate(
            [src, jnp.zeros((ep - e,), jnp.int32)])
        dst = jnp.concatenate(
            [dst, jnp.full((ep - e,), pad_dst_value, jnp.int32)])
    return src, dst, ep


def kernel(freq, flit, op_type, W_freq, b_freq, W_flit, b_flit, W_op, b_op,
           W_fh, b_fh, W_fn, b_fn, Wr1, br1, Wp1, bp1, Wr2, br2, Wp2, bp2,
           Wh1, bh1, Wh2, bh2, Wh3, bh3,
           pass_src, pass_dst, transfer_src, transfer_dst,
           connect_src, connect_dst):
    np_, nr = freq.shape[0], op_type.shape[0]

    i32 = lambda a: a.astype(jnp.int32)
    ps, pd, ep_p = _pad_edges(i32(pass_src), i32(pass_dst), nr)
    cs, cd, ep_c = _pad_edges(i32(connect_src), i32(connect_dst), nr)
    ts, td, ep_t = _pad_edges(i32(transfer_src), i32(transfer_dst), np_)

    row = lambda b: b.reshape(1, -1)
    b_freq_, b_flit_, b_op_, b_fh_, b_fn_ = map(
        row, (b_freq, b_flit, b_op, b_fh, b_fn))
    br1_, bp1_, br2_, bh1_, bh2_, bh3_ = map(
        row, (br1, bp1, br2, bh1, bh2, bh3))

    seg_pass = _make_seg_partials(nr, ep_p, H)
    seg_conn = _make_seg_partials(nr, ep_c, H)
    seg_xfer = _make_seg_chunked(ep_t, H)
    seg_cnt = _make_cnt_hist(ep_t)

    packet0 = _fg_packet(freq, flit, W_freq, b_freq_, W_flit, b_flit_,
                         W_fh, b_fh_)
    router0 = _fg_router(op_type, W_op, b_op_, W_fn, b_fn_)

    h1p = seg_pass(packet0, ps, pd)
    h2p = seg_conn(router0, cs, cd)
    msum = seg_xfer(router0, ts, td)
    cntp = seg_cnt(td)
    cnt1 = cntp.reshape(NC * NS, HR * H).sum(axis=0)[:np_]
    cnt16 = jnp.broadcast_to(cnt1[:, None], (np_, LN))

    router1 = _router_update(router0, h1p, h2p, Wr1, br1_)
    packet1 = _packet_update(packet0, msum[:np_], cnt16, Wp1, bp1_)

    h1p2 = seg_pass(packet1, ps, pd)
    h2p2 = seg_conn(router1, cs, cd)
    router2 = _router_update(router1, h1p2, h2p2, Wr2, br2_)

    return _readout(router2, Wh1, bh1_, Wh2, bh2_, Wh3, bh3_)
